# Initial kernel scaffold; baseline (speedup 1.0000x reference)
import jax
import jax.numpy as jnp
from jax.experimental import pallas as pl


def kernel(x, edge_index, W_enc, b_enc, W_gcn, b_gcn, W_ih, W_hh, b_ih, b_hh, W_pol, b_pol):
    raise NotImplementedError("write your pallas kernel here")


# speedup vs baseline: 14.4686x; 14.4686x over previous
import functools

import jax
import jax.numpy as jnp
from jax import lax
from jax.experimental import pallas as pl
from jax.experimental.pallas import tpu as pltpu
from jax.experimental.pallas import tpu_sc as plsc

N = 10000
E = 320000
H = 128

N_PAD = 10240
CHUNK = 128
NW = 32
EDGE_CHUNKS = (E + NW * CHUNK - 1) // (NW * CHUNK)
E_PAD = NW * CHUNK * EDGE_CHUNKS
ROWS_PER_TILE = N_PAD // 16
ZERO_CHUNKS = ROWS_PER_TILE // CHUNK


def _degree_body(dst_hbm, ones_hbm, zeros_hbm, out_hbm,
                 didx_v, ones_v, cnt_sh, sem):
    c = lax.axis_index("c")
    s = lax.axis_index("s")
    wid = c * 16 + s

    pltpu.sync_copy(ones_hbm, ones_v)
    r0 = pl.multiple_of(s * ROWS_PER_TILE, CHUNK)
    pltpu.sync_copy(zeros_hbm, cnt_sh.at[pl.ds(r0, ROWS_PER_TILE)])
    plsc.subcore_barrier()

    def edge_body(k, _):
        e0 = pl.multiple_of((wid * EDGE_CHUNKS + k) * CHUNK, CHUNK)
        pltpu.sync_copy(dst_hbm.at[pl.ds(e0, CHUNK)], didx_v)
        pltpu.sync_copy(ones_v, cnt_sh.at[didx_v], add=True)
        return 0

    lax.fori_loop(0, EDGE_CHUNKS, edge_body, 0)
    plsc.subcore_barrier()

    pltpu.sync_copy(cnt_sh.at[pl.ds(r0, ROWS_PER_TILE)],
                    out_hbm.at[c, pl.ds(r0, ROWS_PER_TILE)])


@functools.lru_cache(maxsize=None)
def _build_sc_degree():
    mesh = plsc.VectorSubcoreMesh(core_axis_name="c", subcore_axis_name="s")
    return pl.kernel(
        _degree_body,
        out_type=jax.ShapeDtypeStruct((2, N_PAD, H), jnp.float32),
        mesh=mesh,
        scratch_types=[
            pltpu.VMEM((CHUNK,), jnp.int32),
            pltpu.VMEM((CHUNK, H), jnp.float32),
            pltpu.VMEM_SHARED((N_PAD, H), jnp.float32),
            pltpu.SemaphoreType.DMA,
        ],
    )


def _sc_degree(dst_p):
    ones = jnp.ones((CHUNK, H), jnp.float32)
    zeros = jnp.zeros((ROWS_PER_TILE, H), jnp.float32)
    return _build_sc_degree()(dst_p, ones, zeros)


def _aggregate_body(src_hbm, dst_hbm, hw2_hbm, zeros_hbm, out_hbm,
                    sidx_v, didx_v, rows_v, agg_sh, sem):
    c = lax.axis_index("c")
    s = lax.axis_index("s")
    wid = c * 16 + s

    r0 = pl.multiple_of(s * ROWS_PER_TILE, CHUNK)
    pltpu.sync_copy(zeros_hbm, agg_sh.at[pl.ds(r0, ROWS_PER_TILE)])
    plsc.subcore_barrier()

    def edge_body(k, _):
        e0 = pl.multiple_of((wid * EDGE_CHUNKS + k) * CHUNK, CHUNK)
        pltpu.sync_copy(src_hbm.at[pl.ds(e0, CHUNK)], sidx_v)
        pltpu.async_copy(hw2_hbm.at[sidx_v], rows_v, sem).wait()
        pltpu.sync_copy(dst_hbm.at[pl.ds(e0, CHUNK)], didx_v)
        pltpu.sync_copy(rows_v, agg_sh.at[didx_v], add=True)
        return 0

    lax.fori_loop(0, EDGE_CHUNKS, edge_body, 0)
    plsc.subcore_barrier()

    pltpu.sync_copy(agg_sh.at[pl.ds(r0, ROWS_PER_TILE)],
                    out_hbm.at[c, pl.ds(r0, ROWS_PER_TILE)])


@functools.lru_cache(maxsize=None)
def _build_sc_aggregate():
    mesh = plsc.VectorSubcoreMesh(core_axis_name="c", subcore_axis_name="s")
    return pl.kernel(
        _aggregate_body,
        out_type=jax.ShapeDtypeStruct((2, N_PAD, H), jnp.float32),
        mesh=mesh,
        scratch_types=[
            pltpu.VMEM((CHUNK,), jnp.int32),
            pltpu.VMEM((CHUNK,), jnp.int32),
            pltpu.VMEM((CHUNK, H), jnp.float32),
            pltpu.VMEM_SHARED((N_PAD, H), jnp.float32),
            pltpu.SemaphoreType.DMA,
        ],
    )


def _sc_aggregate(src_p, dst_p, hw2_p):
    zeros = jnp.zeros((ROWS_PER_TILE, H), jnp.float32)
    return _build_sc_aggregate()(src_p, dst_p, hw2_p, zeros)


BLK = 1000


def _k1_body(cnt_ref, x_ref, wenc_t_ref, benc_ref, wgcn_t_ref,
             hw2_ref, dinv_ref):
    deg = cnt_ref[0, :, 0:1] + cnt_ref[1, :, 0:1] + 1.0
    dinv = lax.rsqrt(deg)
    h = jnp.dot(x_ref[...], wenc_t_ref[...],
                preferred_element_type=jnp.float32) + benc_ref[...]
    hw = jnp.dot(h, wgcn_t_ref[...], preferred_element_type=jnp.float32)
    hw2_ref[...] = hw * dinv
    dinv_ref[...] = dinv


def _run_k1(cnt, x, wenc_t, benc, wgcn_t):
    return pl.pallas_call(
        _k1_body,
        grid=(N // BLK,),
        in_specs=[
            pl.BlockSpec((2, BLK, 1), lambda i: (0, i, 0)),
            pl.BlockSpec((BLK, H), lambda i: (i, 0)),
            pl.BlockSpec((H, H), lambda i: (0, 0)),
            pl.BlockSpec((1, H), lambda i: (0, 0)),
            pl.BlockSpec((H, H), lambda i: (0, 0)),
        ],
        out_specs=[
            pl.BlockSpec((BLK, H), lambda i: (i, 0)),
            pl.BlockSpec((BLK, 1), lambda i: (i, 0)),
        ],
        out_shape=[
            jax.ShapeDtypeStruct((N, H), jnp.float32),
            jax.ShapeDtypeStruct((N, 1), jnp.float32),
        ],
    )(cnt, x, wenc_t, benc, wgcn_t)


def _k2_body(agg_ref, hw2_ref, dinv_ref, bgcn_ref, wih_t_ref, bih_ref,
             gi_ref):
    a = agg_ref[0] + agg_ref[1] + hw2_ref[...]
    gcn = a * dinv_ref[...] + bgcn_ref[...]
    gi_ref[...] = jnp.dot(gcn, wih_t_ref[...],
                          preferred_element_type=jnp.float32) + bih_ref[...]


def _run_k2(agg, hw2, dinv, bgcn, wih_t, bih):
    return pl.pallas_call(
        _k2_body,
        grid=(N // BLK,),
        in_specs=[
            pl.BlockSpec((2, BLK, H), lambda i: (0, i, 0)),
            pl.BlockSpec((BLK, H), lambda i: (i, 0)),
            pl.BlockSpec((BLK, 1), lambda i: (i, 0)),
            pl.BlockSpec((1, H), lambda i: (0, 0)),
            pl.BlockSpec((H, 3 * H), lambda i: (0, 0)),
            pl.BlockSpec((1, 3 * H), lambda i: (0, 0)),
        ],
        out_specs=pl.BlockSpec((BLK, 3 * H), lambda i: (i, 0)),
        out_shape=jax.ShapeDtypeStruct((N, 3 * H), jnp.float32),
    )(agg, hw2, dinv, bgcn, wih_t, bih)


def _k3_body(gi_ref, whh_t_ref, bhh_ref, wpol_t_ref, bpol_ref,
             out_ref, hcarry_ref, hbuf_ref):
    pid = pl.program_id(0)

    @pl.when(pid == 0)
    def _():
        hcarry_ref[...] = jnp.zeros((8, H), jnp.float32)

    whh_t = whh_t_ref[...]
    bhh = bhh_ref[...]

    def step(i, hprev):
        gi = gi_ref[pl.ds(i, 1), :]
        gh = jnp.dot(hprev, whh_t,
                     preferred_element_type=jnp.float32) + bhh
        g = gi + gh
        r = jax.nn.sigmoid(g[:, 0:H])
        z = jax.nn.sigmoid(g[:, H:2 * H])
        nn_ = jnp.tanh(gi[:, 2 * H:3 * H] + gh[:, 2 * H:3 * H] * r)
        hnew = (1.0 - z) * nn_ + z * hprev
        hbuf_ref[pl.ds(i, 1), :] = hnew[0:1, :]
        return hnew

    h0 = hcarry_ref[...]
    hfin = lax.fori_loop(0, BLK, step, h0)
    hcarry_ref[...] = hfin
    out_ref[...] = jnp.dot(hbuf_ref[...], wpol_t_ref[...],
                           preferred_element_type=jnp.float32) + bpol_ref[...]


def _run_k3(gi, whh_t, bhh, wpol_t, bpol):
    return pl.pallas_call(
        _k3_body,
        grid=(N // BLK,),
        in_specs=[
            pl.BlockSpec((BLK, 3 * H), lambda i: (i, 0)),
            pl.BlockSpec((H, 3 * H), lambda i: (0, 0)),
            pl.BlockSpec((1, 3 * H), lambda i: (0, 0)),
            pl.BlockSpec((H, 1), lambda i: (0, 0)),
            pl.BlockSpec((1, 1), lambda i: (0, 0)),
        ],
        out_specs=pl.BlockSpec((BLK, 1), lambda i: (i, 0)),
        out_shape=jax.ShapeDtypeStruct((N, 1), jnp.float32),
        scratch_shapes=[
            pltpu.VMEM((8, H), jnp.float32),
            pltpu.VMEM((BLK, H), jnp.float32),
        ],
    )(gi, whh_t, bhh, wpol_t, bpol)


@jax.jit
def kernel(x, edge_index, W_enc, b_enc, W_gcn, b_gcn, W_ih, W_hh, b_ih, b_hh,
           W_pol, b_pol):
    src = edge_index[0]
    dst = edge_index[1]
    pad = E_PAD - E
    src_p = jnp.concatenate([src, jnp.full((pad,), N, jnp.int32)])
    dst_p = jnp.concatenate([dst, jnp.full((pad,), N, jnp.int32)])

    cnt = _sc_degree(dst_p)

    hw2, dinv = _run_k1(cnt[:, :N, :1], x, W_enc.T, b_enc.reshape(1, H),
                        W_gcn.T)

    hw2_p = jnp.concatenate([hw2, jnp.zeros((N_PAD - N, H), jnp.float32)])
    agg = _sc_aggregate(src_p, dst_p, hw2_p)

    gi = _run_k2(agg[:, :N, :], hw2, dinv, b_gcn.reshape(1, H), W_ih.T,
                 b_ih.reshape(1, 3 * H))

    scores = _run_k3(gi, W_hh.T, b_hh.reshape(1, 3 * H), W_pol.T,
                     b_pol.reshape(1, 1))
    return scores[:, 0]
